# Initial kernel scaffold; baseline (speedup 1.0000x reference)
import functools

import jax
import jax.numpy as jnp
from jax import lax
from jax.experimental import pallas as pl
from jax.experimental.pallas import tpu as pltpu
from jax.experimental.pallas import tpu_sc as plsc

N_NODES = 10000
N_EDGES = 320000
D_FEAT = 128
NUM_LAYERS = 3
NUM_OUTPUTS = 1000

NC = 2
NS = 16
NW = NC * NS
CHUNK = 128
N_CHUNKS = N_EDGES // CHUNK
NODES_PER_TILE = N_NODES // NS
BCHUNK = 100
NB_CHUNKS = N_NODES // BCHUNK
FB = D_FEAT // 16


def _mesh():
    return plsc.VectorSubcoreMesh(
        core_axis_name="c", subcore_axis_name="s", num_cores=NC,
        num_subcores=NS)


@functools.partial(
    pl.kernel,
    out_type=jax.ShapeDtypeStruct((NC, N_NODES, D_FEAT), jnp.float32),
    mesh=_mesh(),
    scratch_types=[
        pltpu.VMEM_SHARED((N_NODES, D_FEAT), jnp.float32),
        pltpu.VMEM((CHUNK, D_FEAT), jnp.float32),
        pltpu.VMEM((CHUNK,), jnp.int32),
        pltpu.VMEM((CHUNK,), jnp.int32),
        pltpu.VMEM((CHUNK,), jnp.float32),
        pltpu.SemaphoreType.DMA,
    ],
)
def _phase_a(vals_hbm, src_hbm, dst_hbm, w_hbm, zeros_hbm, part_hbm,
             acc_sp, rows_v, src_v, dst_v, w_v, sem):
    cid = lax.axis_index("c")
    sid = lax.axis_index("s")
    wid = sid * NC + cid

    pltpu.sync_copy(zeros_hbm, acc_sp.at[pl.ds(sid * NODES_PER_TILE,
                                               NODES_PER_TILE)])
    plsc.subcore_barrier()

    rem = N_CHUNKS - (N_CHUNKS // NW) * NW
    cnt = jnp.where(wid < rem, N_CHUNKS // NW + 1, N_CHUNKS // NW)

    def chunk_body(k, carry):
        off = (wid + NW * k) * CHUNK
        pltpu.sync_copy(src_hbm.at[pl.ds(off, CHUNK)], src_v)
        pltpu.sync_copy(dst_hbm.at[pl.ds(off, CHUNK)], dst_v)
        pltpu.sync_copy(w_hbm.at[pl.ds(off, CHUNK)], w_v)
        pltpu.async_copy(vals_hbm.at[src_v], rows_v, sem).wait()

        def edge_body(e, c):
            bidx = jnp.full((16,), e, jnp.int32)
            wv = plsc.load_gather(w_v, [bidx])
            for j in range(FB):
                rows_v[e, pl.ds(j * 16, 16)] = (
                    rows_v[e, pl.ds(j * 16, 16)] * wv)
            return c

        lax.fori_loop(0, CHUNK, edge_body, 0)
        pltpu.sync_copy(rows_v, acc_sp.at[dst_v], add=True)
        return carry

    lax.fori_loop(0, cnt, chunk_body, 0)
    plsc.subcore_barrier()
    pltpu.sync_copy(
        acc_sp.at[pl.ds(sid * NODES_PER_TILE, NODES_PER_TILE)],
        part_hbm.at[cid, pl.ds(sid * NODES_PER_TILE, NODES_PER_TILE)])


@functools.partial(
    pl.kernel,
    out_type=jax.ShapeDtypeStruct((N_NODES, D_FEAT), jnp.float32),
    mesh=_mesh(),
    scratch_types=[
        pltpu.VMEM((BCHUNK, D_FEAT), jnp.float32),
        pltpu.VMEM((BCHUNK, D_FEAT), jnp.float32),
    ],
)
def _phase_b(part_hbm, out_hbm, a_v, b_v):
    cid = lax.axis_index("c")
    sid = lax.axis_index("s")
    wid = sid * NC + cid

    rem = NB_CHUNKS - (NB_CHUNKS // NW) * NW
    cnt = jnp.where(wid < rem, NB_CHUNKS // NW + 1, NB_CHUNKS // NW)

    def chunk_body(k, carry):
        off = (wid + NW * k) * BCHUNK
        pltpu.sync_copy(part_hbm.at[0, pl.ds(off, BCHUNK)], a_v)
        pltpu.sync_copy(part_hbm.at[1, pl.ds(off, BCHUNK)], b_v)

        def node_body(i, c):
            vs = [a_v[i, pl.ds(j * 16, 16)] + b_v[i, pl.ds(j * 16, 16)]
                  for j in range(FB)]
            m = vs[0]
            for j in range(1, FB):
                m = jnp.maximum(m, vs[j])
            ms = jnp.max(m)
            es = [jnp.exp(v - ms) for v in vs]
            s = es[0]
            for j in range(1, FB):
                s = s + es[j]
            r = 1.0 / jnp.sum(s)
            for j in range(FB):
                a_v[i, pl.ds(j * 16, 16)] = es[j] * r
            return c

        lax.fori_loop(0, BCHUNK, node_body, 0)
        pltpu.sync_copy(a_v, out_hbm.at[pl.ds(off, BCHUNK)])
        return carry

    lax.fori_loop(0, cnt, chunk_body, 0)


def kernel(x, edge_index, edge_weight):
    src = edge_index[0].astype(jnp.int32)
    dst = edge_index[1].astype(jnp.int32)
    w = edge_weight.astype(jnp.float32)
    zeros = jnp.zeros((NODES_PER_TILE, D_FEAT), jnp.float32)
    vals = x
    for _ in range(NUM_LAYERS):
        part = _phase_a(vals, src, dst, w, zeros)
        vals = _phase_b(part)
    return vals[N_NODES - NUM_OUTPUTS:]


# speedup vs baseline: 4.7749x; 4.7749x over previous
import functools

import jax
import jax.numpy as jnp
from jax import lax
from jax.experimental import pallas as pl
from jax.experimental.pallas import tpu as pltpu
from jax.experimental.pallas import tpu_sc as plsc

N_NODES = 10000
N_EDGES = 320000
D_FEAT = 128
NUM_LAYERS = 3
NUM_OUTPUTS = 1000

NC = 2
NS = 16
NW = NC * NS
CHUNK = 128
N_CHUNKS = N_EDGES // CHUNK
NPT = 624
TAIL = N_NODES - NPT * NS
BCHUNK = 80
NB_CHUNKS = N_NODES // BCHUNK
FB = D_FEAT // 16


def _shuffle_xor(v, shift):
    idx = jnp.reshape(
        jax.lax.iota(jnp.int32, 16) ^ jnp.int32(shift), (16, 1))
    return lax.gather(
        v, idx,
        dimension_numbers=lax.GatherDimensionNumbers(
            offset_dims=(), collapsed_slice_dims=(0,), start_index_map=(0,)),
        slice_sizes=(1,), mode=lax.GatherScatterMode.PROMISE_IN_BOUNDS)


def _mesh():
    return plsc.VectorSubcoreMesh(
        core_axis_name="c", subcore_axis_name="s", num_cores=NC,
        num_subcores=NS)


@functools.partial(
    pl.kernel,
    out_type=jax.ShapeDtypeStruct((NC, N_NODES, D_FEAT), jnp.float32),
    mesh=_mesh(),
    scratch_types=[
        pltpu.VMEM_SHARED((N_NODES, D_FEAT), jnp.float32),
        pltpu.VMEM((CHUNK, D_FEAT), jnp.float32),
        pltpu.VMEM((CHUNK,), jnp.int32),
        pltpu.VMEM((CHUNK,), jnp.int32),
        pltpu.VMEM((CHUNK,), jnp.float32),
        pltpu.SemaphoreType.DMA,
    ],
)
def _phase_a(vals_hbm, src_hbm, dst_hbm, w_hbm, zeros_hbm, part_hbm,
             acc_sp, rows_v, src_v, dst_v, w_v, sem):
    cid = lax.axis_index("c")
    sid = lax.axis_index("s")
    wid = sid * NC + cid

    pltpu.sync_copy(zeros_hbm.at[pl.ds(0, NPT)],
                    acc_sp.at[pl.ds(sid * NPT, NPT)])

    @pl.when(sid == NS - 1)
    def _zero_tail():
        pltpu.sync_copy(zeros_hbm.at[pl.ds(0, TAIL)],
                        acc_sp.at[pl.ds(NPT * NS, TAIL)])

    plsc.subcore_barrier()

    rem = N_CHUNKS - (N_CHUNKS // NW) * NW
    cnt = jnp.where(wid < rem, N_CHUNKS // NW + 1, N_CHUNKS // NW)

    def chunk_body(k, carry):
        off = (wid + NW * k) * CHUNK
        pltpu.sync_copy(src_hbm.at[pl.ds(off, CHUNK)], src_v)
        pltpu.sync_copy(dst_hbm.at[pl.ds(off, CHUNK)], dst_v)
        pltpu.sync_copy(w_hbm.at[pl.ds(off, CHUNK)], w_v)
        pltpu.async_copy(vals_hbm.at[src_v], rows_v, sem).wait()

        def grp_body(g, c):
            base = g * 16
            wblk = w_v[pl.ds(base, 16)]
            for l in range(16):
                e = base + l
                idx = jnp.full((16, 1), l, jnp.int32)
                wv = lax.gather(
                    wblk, idx,
                    dimension_numbers=lax.GatherDimensionNumbers(
                        offset_dims=(), collapsed_slice_dims=(0,),
                        start_index_map=(0,)),
                    slice_sizes=(1,),
                    mode=lax.GatherScatterMode.PROMISE_IN_BOUNDS)
                for j in range(FB):
                    rows_v[e, pl.ds(j * 16, 16)] = (
                        rows_v[e, pl.ds(j * 16, 16)] * wv)
            return c

        lax.fori_loop(0, CHUNK // 16, grp_body, 0)
        pltpu.sync_copy(rows_v, acc_sp.at[dst_v], add=True)
        return carry

    lax.fori_loop(0, cnt, chunk_body, 0)
    plsc.subcore_barrier()
    pltpu.sync_copy(acc_sp.at[pl.ds(sid * NPT, NPT)],
                    part_hbm.at[cid, pl.ds(sid * NPT, NPT)])

    @pl.when(sid == NS - 1)
    def _dump_tail():
        pltpu.sync_copy(acc_sp.at[pl.ds(NPT * NS, TAIL)],
                        part_hbm.at[cid, pl.ds(NPT * NS, TAIL)])


@functools.partial(
    pl.kernel,
    out_type=jax.ShapeDtypeStruct((N_NODES, D_FEAT), jnp.float32),
    mesh=_mesh(),
    scratch_types=[
        pltpu.VMEM((BCHUNK, D_FEAT), jnp.float32),
        pltpu.VMEM((BCHUNK, D_FEAT), jnp.float32),
    ],
)
def _phase_b(part_hbm, out_hbm, a_v, b_v):
    cid = lax.axis_index("c")
    sid = lax.axis_index("s")
    wid = sid * NC + cid

    rem = NB_CHUNKS - (NB_CHUNKS // NW) * NW
    cnt = jnp.where(wid < rem, NB_CHUNKS // NW + 1, NB_CHUNKS // NW)

    def chunk_body(k, carry):
        off = (wid + NW * k) * BCHUNK
        pltpu.sync_copy(part_hbm.at[0, pl.ds(off, BCHUNK)], a_v)
        pltpu.sync_copy(part_hbm.at[1, pl.ds(off, BCHUNK)], b_v)

        def node_body(i, c):
            vs = [a_v[i, pl.ds(j * 16, 16)] + b_v[i, pl.ds(j * 16, 16)]
                  for j in range(FB)]
            m = vs[0]
            for j in range(1, FB):
                m = jnp.maximum(m, vs[j])
            for sh in (8, 4, 2, 1):
                m = jnp.maximum(m, _shuffle_xor(m, sh))
            es = [jnp.exp(v - m) for v in vs]
            s = es[0]
            for j in range(1, FB):
                s = s + es[j]
            for sh in (8, 4, 2, 1):
                s = s + _shuffle_xor(s, sh)
            r = 1.0 / s
            for j in range(FB):
                a_v[i, pl.ds(j * 16, 16)] = es[j] * r
            return c

        lax.fori_loop(0, BCHUNK, node_body, 0)
        pltpu.sync_copy(a_v, out_hbm.at[pl.ds(off, BCHUNK)])
        return carry

    lax.fori_loop(0, cnt, chunk_body, 0)


def kernel(x, edge_index, edge_weight):
    src = edge_index[0].astype(jnp.int32)
    dst = edge_index[1].astype(jnp.int32)
    w = edge_weight.astype(jnp.float32)
    zeros = jnp.zeros((NPT, D_FEAT), jnp.float32)
    vals = x
    for _ in range(NUM_LAYERS):
        part = _phase_a(vals, src, dst, w, zeros)
        vals = _phase_b(part)
    return vals[N_NODES - NUM_OUTPUTS:]


# speedup vs baseline: 8.8901x; 1.8618x over previous
import functools

import jax
import jax.numpy as jnp
from jax import lax
from jax.experimental import pallas as pl
from jax.experimental.pallas import tpu as pltpu
from jax.experimental.pallas import tpu_sc as plsc

N_NODES = 10000
N_EDGES = 320000
D_FEAT = 128
NUM_LAYERS = 3
NUM_OUTPUTS = 1000

NC = 2
NS = 16
NW = NC * NS
CHUNK = 128
N_CHUNKS = N_EDGES // CHUNK
NPT = 624
TAIL = N_NODES - NPT * NS
BCHUNK = 80
NB_CHUNKS = N_NODES // BCHUNK
FB = D_FEAT // 16


def _shuffle_xor(v, shift):
    idx = jnp.reshape(
        jax.lax.iota(jnp.int32, 16) ^ jnp.int32(shift), (16, 1))
    return lax.gather(
        v, idx,
        dimension_numbers=lax.GatherDimensionNumbers(
            offset_dims=(), collapsed_slice_dims=(0,), start_index_map=(0,)),
        slice_sizes=(1,), mode=lax.GatherScatterMode.PROMISE_IN_BOUNDS)


def _mesh():
    return plsc.VectorSubcoreMesh(
        core_axis_name="c", subcore_axis_name="s", num_cores=NC,
        num_subcores=NS)


def _lane_bcast(wblk, l):
    idx = jnp.full((16, 1), l, jnp.int32)
    return lax.gather(
        wblk, idx,
        dimension_numbers=lax.GatherDimensionNumbers(
            offset_dims=(), collapsed_slice_dims=(0,), start_index_map=(0,)),
        slice_sizes=(1,), mode=lax.GatherScatterMode.PROMISE_IN_BOUNDS)


@functools.partial(
    pl.kernel,
    out_type=jax.ShapeDtypeStruct((NC, N_NODES, D_FEAT), jnp.float32),
    mesh=_mesh(),
    scratch_types=[
        pltpu.VMEM_SHARED((N_NODES, D_FEAT), jnp.float32),
        pltpu.VMEM((2, CHUNK, D_FEAT), jnp.float32),
        pltpu.VMEM((2, 2, CHUNK), jnp.int32),
        pltpu.VMEM((2, CHUNK), jnp.float32),
        pltpu.SemaphoreType.DMA,
        pltpu.SemaphoreType.DMA,
    ],
)
def _phase_a(vals_hbm, edges_hbm, w_hbm, zeros_hbm, part_hbm,
             acc_sp, rows_v, e_v, w_v, semI, semG):
    cid = lax.axis_index("c")
    sid = lax.axis_index("s")
    wid = sid * NC + cid

    pltpu.sync_copy(zeros_hbm.at[pl.ds(0, NPT)],
                    acc_sp.at[pl.ds(sid * NPT, NPT)])

    @pl.when(sid == NS - 1)
    def _zero_tail():
        pltpu.sync_copy(zeros_hbm.at[pl.ds(0, TAIL)],
                        acc_sp.at[pl.ds(NPT * NS, TAIL)])

    plsc.subcore_barrier()

    rem = N_CHUNKS - (N_CHUNKS // NW) * NW
    cnt = jnp.where(wid < rem, N_CHUNKS // NW + 1, N_CHUNKS // NW)

    def coff(k):
        return (wid + NW * k) * CHUNK

    def stage_idx(k, b):
        pltpu.async_copy(edges_hbm.at[:, pl.ds(coff(k), CHUNK)],
                         e_v.at[b], semI)
        pltpu.async_copy(w_hbm.at[pl.ds(coff(k), CHUNK)], w_v.at[b], semI)

    def wait_idx(k, b):
        pltpu.make_async_copy(edges_hbm.at[:, pl.ds(coff(k), CHUNK)],
                              e_v.at[b], semI).wait()
        pltpu.make_async_copy(w_hbm.at[pl.ds(coff(k), CHUNK)],
                              w_v.at[b], semI).wait()

    def start_gather(b):
        return pltpu.async_copy(vals_hbm.at[e_v.at[b, 0]], rows_v.at[b],
                                semG)

    stage_idx(0, 0)
    wait_idx(0, 0)
    start_gather(0)

    @pl.when(cnt > 1)
    def _prefetch1():
        stage_idx(1, 1)

    def chunk_body(k, carry):
        b = lax.rem(k, 2)

        def scale_scatter(b):
            pltpu.make_async_copy(vals_hbm.at[e_v.at[b, 0]], rows_v.at[b],
                                  semG).wait()

            @pl.when(k + 1 < cnt)
            def _next_gather():
                nb = 1 - b
                wait_idx(k + 1, nb)
                start_gather(nb)

            def grp_body(g, c):
                base = g * 16
                wblk = w_v[b, pl.ds(base, 16)]
                for l in range(16):
                    e = base + l
                    wv = _lane_bcast(wblk, l)
                    for j in range(FB):
                        rows_v[b, e, pl.ds(j * 16, 16)] = (
                            rows_v[b, e, pl.ds(j * 16, 16)] * wv)
                return c

            lax.fori_loop(0, CHUNK // 16, grp_body, 0)
            pltpu.sync_copy(rows_v.at[b], acc_sp.at[e_v.at[b, 1]], add=True)

            @pl.when(k + 2 < cnt)
            def _next_idx():
                stage_idx(k + 2, b)

        @pl.when(b == 0)
        def _b0():
            scale_scatter(0)

        @pl.when(b == 1)
        def _b1():
            scale_scatter(1)

        return carry

    lax.fori_loop(0, cnt, chunk_body, 0)
    plsc.subcore_barrier()
    pltpu.sync_copy(acc_sp.at[pl.ds(sid * NPT, NPT)],
                    part_hbm.at[cid, pl.ds(sid * NPT, NPT)])

    @pl.when(sid == NS - 1)
    def _dump_tail():
        pltpu.sync_copy(acc_sp.at[pl.ds(NPT * NS, TAIL)],
                        part_hbm.at[cid, pl.ds(NPT * NS, TAIL)])


@functools.partial(
    pl.kernel,
    out_type=jax.ShapeDtypeStruct((N_NODES, D_FEAT), jnp.float32),
    mesh=_mesh(),
    scratch_types=[
        pltpu.VMEM((BCHUNK, D_FEAT), jnp.float32),
        pltpu.VMEM((BCHUNK, D_FEAT), jnp.float32),
    ],
)
def _phase_b(part_hbm, out_hbm, a_v, b_v):
    cid = lax.axis_index("c")
    sid = lax.axis_index("s")
    wid = sid * NC + cid

    rem = NB_CHUNKS - (NB_CHUNKS // NW) * NW
    cnt = jnp.where(wid < rem, NB_CHUNKS // NW + 1, NB_CHUNKS // NW)

    def chunk_body(k, carry):
        off = (wid + NW * k) * BCHUNK
        pltpu.sync_copy(part_hbm.at[0, pl.ds(off, BCHUNK)], a_v)
        pltpu.sync_copy(part_hbm.at[1, pl.ds(off, BCHUNK)], b_v)

        def node_body(i, c):
            vs = [a_v[i, pl.ds(j * 16, 16)] + b_v[i, pl.ds(j * 16, 16)]
                  for j in range(FB)]
            m = vs[0]
            for j in range(1, FB):
                m = jnp.maximum(m, vs[j])
            for sh in (8, 4, 2, 1):
                m = jnp.maximum(m, _shuffle_xor(m, sh))
            es = [jnp.exp(v - m) for v in vs]
            s = es[0]
            for j in range(1, FB):
                s = s + es[j]
            for sh in (8, 4, 2, 1):
                s = s + _shuffle_xor(s, sh)
            r = 1.0 / s
            for j in range(FB):
                a_v[i, pl.ds(j * 16, 16)] = es[j] * r
            return c

        lax.fori_loop(0, BCHUNK, node_body, 0)
        pltpu.sync_copy(a_v, out_hbm.at[pl.ds(off, BCHUNK)])
        return carry

    lax.fori_loop(0, cnt, chunk_body, 0)


def kernel(x, edge_index, edge_weight):
    src = edge_index[0].astype(jnp.int32)
    dst = edge_index[1].astype(jnp.int32)
    w = edge_weight.astype(jnp.float32)
    edges = jnp.stack([src, dst])
    zeros = jnp.zeros((NPT, D_FEAT), jnp.float32)
    vals = x
    for _ in range(NUM_LAYERS):
        part = _phase_a(vals, edges, w, zeros)
        vals = _phase_b(part)
    return vals[N_NODES - NUM_OUTPUTS:]


# speedup vs baseline: 10.1967x; 1.1470x over previous
import functools

import jax
import jax.numpy as jnp
from jax import lax
from jax.experimental import pallas as pl
from jax.experimental.pallas import tpu as pltpu
from jax.experimental.pallas import tpu_sc as plsc

N_NODES = 10000
N_EDGES = 320000
D_FEAT = 128
NUM_LAYERS = 3
NUM_OUTPUTS = 1000

NC = 2
NS = 16
NW = NC * NS
CHUNK = 128
N_CHUNKS = N_EDGES // CHUNK
NPT = 624
TAIL = N_NODES - NPT * NS
BCHUNK = 80
NB_CHUNKS = N_NODES // BCHUNK
FB = D_FEAT // 16


def _shuffle_xor(v, shift):
    idx = jnp.reshape(
        jax.lax.iota(jnp.int32, 16) ^ jnp.int32(shift), (16, 1))
    return lax.gather(
        v, idx,
        dimension_numbers=lax.GatherDimensionNumbers(
            offset_dims=(), collapsed_slice_dims=(0,), start_index_map=(0,)),
        slice_sizes=(1,), mode=lax.GatherScatterMode.PROMISE_IN_BOUNDS)


def _mesh():
    return plsc.VectorSubcoreMesh(
        core_axis_name="c", subcore_axis_name="s", num_cores=NC,
        num_subcores=NS)


def _lane_bcast(wblk, l):
    idx = jnp.full((16, 1), l, jnp.int32)
    return lax.gather(
        wblk, idx,
        dimension_numbers=lax.GatherDimensionNumbers(
            offset_dims=(), collapsed_slice_dims=(0,), start_index_map=(0,)),
        slice_sizes=(1,), mode=lax.GatherScatterMode.PROMISE_IN_BOUNDS)


@functools.partial(
    pl.kernel,
    out_type=jax.ShapeDtypeStruct((NC, N_NODES, D_FEAT), jnp.float32),
    mesh=_mesh(),
    scratch_types=[
        pltpu.VMEM_SHARED((N_NODES, D_FEAT), jnp.float32),
        pltpu.VMEM((3, CHUNK, D_FEAT), jnp.float32),
        pltpu.VMEM((3, 2, CHUNK), jnp.int32),
        pltpu.VMEM((3, CHUNK), jnp.float32),
        pltpu.SemaphoreType.DMA,
        pltpu.SemaphoreType.DMA,
        pltpu.SemaphoreType.DMA,
    ],
)
def _phase_a(vals_hbm, edges_hbm, w_hbm, zeros_hbm, part_hbm,
             acc_sp, rows_v, e_v, w_v, semI, semG, semS):
    cid = lax.axis_index("c")
    sid = lax.axis_index("s")
    wid = sid * NC + cid

    pltpu.sync_copy(zeros_hbm.at[pl.ds(0, NPT)],
                    acc_sp.at[pl.ds(sid * NPT, NPT)])

    @pl.when(sid == NS - 1)
    def _zero_tail():
        pltpu.sync_copy(zeros_hbm.at[pl.ds(0, TAIL)],
                        acc_sp.at[pl.ds(NPT * NS, TAIL)])

    plsc.subcore_barrier()

    rem = N_CHUNKS - (N_CHUNKS // NW) * NW
    cnt = jnp.where(wid < rem, N_CHUNKS // NW + 1, N_CHUNKS // NW)

    def coff(k):
        return (wid + NW * k) * CHUNK

    def stage_idx(k, b):
        pltpu.async_copy(edges_hbm.at[:, pl.ds(coff(k), CHUNK)],
                         e_v.at[b], semI)
        pltpu.async_copy(w_hbm.at[pl.ds(coff(k), CHUNK)], w_v.at[b], semI)

    def wait_idx(k, b):
        pltpu.make_async_copy(edges_hbm.at[:, pl.ds(coff(k), CHUNK)],
                              e_v.at[b], semI).wait()
        pltpu.make_async_copy(w_hbm.at[pl.ds(coff(k), CHUNK)],
                              w_v.at[b], semI).wait()

    def start_gather(b):
        pltpu.async_copy(vals_hbm.at[e_v.at[b, 0]], rows_v.at[b], semG)

    def wait_gather(b):
        pltpu.make_async_copy(vals_hbm.at[e_v.at[b, 0]], rows_v.at[b],
                              semG).wait()

    def start_scatter(b):
        pltpu.async_copy(rows_v.at[b], acc_sp.at[e_v.at[b, 1]], semS,
                         add=True)

    def wait_scatter(b):
        pltpu.make_async_copy(rows_v.at[b], acc_sp.at[e_v.at[b, 1]],
                              semS).wait()

    stage_idx(0, 0)
    wait_idx(0, 0)
    start_gather(0)

    @pl.when(cnt > 1)
    def _prefetch1():
        stage_idx(1, 1)

    def chunk_body(k, carry):
        b3 = lax.rem(k, 3)

        def run(b, nb, pb):
            wait_gather(b)

            @pl.when(k + 1 < cnt)
            def _next_gather():
                wait_idx(k + 1, nb)
                start_gather(nb)

            def grp_body(g, c):
                base = g * 16
                wblk = w_v[b, pl.ds(base, 16)]
                for l in range(16):
                    e = base + l
                    wv = _lane_bcast(wblk, l)
                    for j in range(FB):
                        rows_v[b, e, pl.ds(j * 16, 16)] = (
                            rows_v[b, e, pl.ds(j * 16, 16)] * wv)
                return c

            lax.fori_loop(0, CHUNK // 16, grp_body, 0)

            @pl.when(k > 0)
            def _drain_prev_scatter():
                wait_scatter(pb)

            start_scatter(b)

            @pl.when(k + 2 < cnt)
            def _next_idx():
                stage_idx(k + 2, pb)

        @pl.when(b3 == 0)
        def _b0():
            run(0, 1, 2)

        @pl.when(b3 == 1)
        def _b1():
            run(1, 2, 0)

        @pl.when(b3 == 2)
        def _b2():
            run(2, 0, 1)

        return carry

    lax.fori_loop(0, cnt, chunk_body, 0)
    lb = lax.rem(cnt - 1, 3)
    for b in range(3):
        @pl.when(lb == b)
        def _drain_last(b=b):
            wait_scatter(b)

    plsc.subcore_barrier()
    pltpu.sync_copy(acc_sp.at[pl.ds(sid * NPT, NPT)],
                    part_hbm.at[cid, pl.ds(sid * NPT, NPT)])

    @pl.when(sid == NS - 1)
    def _dump_tail():
        pltpu.sync_copy(acc_sp.at[pl.ds(NPT * NS, TAIL)],
                        part_hbm.at[cid, pl.ds(NPT * NS, TAIL)])


@functools.partial(
    pl.kernel,
    out_type=jax.ShapeDtypeStruct((N_NODES, D_FEAT), jnp.float32),
    mesh=_mesh(),
    scratch_types=[
        pltpu.VMEM((BCHUNK, D_FEAT), jnp.float32),
        pltpu.VMEM((BCHUNK, D_FEAT), jnp.float32),
    ],
)
def _phase_b(part_hbm, out_hbm, a_v, b_v):
    cid = lax.axis_index("c")
    sid = lax.axis_index("s")
    wid = sid * NC + cid

    rem = NB_CHUNKS - (NB_CHUNKS // NW) * NW
    cnt = jnp.where(wid < rem, NB_CHUNKS // NW + 1, NB_CHUNKS // NW)

    def chunk_body(k, carry):
        off = (wid + NW * k) * BCHUNK
        pltpu.sync_copy(part_hbm.at[0, pl.ds(off, BCHUNK)], a_v)
        pltpu.sync_copy(part_hbm.at[1, pl.ds(off, BCHUNK)], b_v)

        def node_body(i, c):
            vs = [a_v[i, pl.ds(j * 16, 16)] + b_v[i, pl.ds(j * 16, 16)]
                  for j in range(FB)]
            m = vs[0]
            for j in range(1, FB):
                m = jnp.maximum(m, vs[j])
            for sh in (8, 4, 2, 1):
                m = jnp.maximum(m, _shuffle_xor(m, sh))
            es = [jnp.exp(v - m) for v in vs]
            s = es[0]
            for j in range(1, FB):
                s = s + es[j]
            for sh in (8, 4, 2, 1):
                s = s + _shuffle_xor(s, sh)
            r = 1.0 / s
            for j in range(FB):
                a_v[i, pl.ds(j * 16, 16)] = es[j] * r
            return c

        lax.fori_loop(0, BCHUNK, node_body, 0)
        pltpu.sync_copy(a_v, out_hbm.at[pl.ds(off, BCHUNK)])
        return carry

    lax.fori_loop(0, cnt, chunk_body, 0)


def kernel(x, edge_index, edge_weight):
    src = edge_index[0].astype(jnp.int32)
    dst = edge_index[1].astype(jnp.int32)
    w = edge_weight.astype(jnp.float32)
    edges = jnp.stack([src, dst])
    zeros = jnp.zeros((NPT, D_FEAT), jnp.float32)
    vals = x
    for _ in range(NUM_LAYERS):
        part = _phase_a(vals, edges, w, zeros)
        vals = _phase_b(part)
    return vals[N_NODES - NUM_OUTPUTS:]


# speedup vs baseline: 11.4217x; 1.1201x over previous
import functools

import jax
import jax.numpy as jnp
from jax import lax
from jax.experimental import pallas as pl
from jax.experimental.pallas import tpu as pltpu
from jax.experimental.pallas import tpu_sc as plsc

N_NODES = 10000
N_EDGES = 320000
D_FEAT = 128
NUM_LAYERS = 3
NUM_OUTPUTS = 1000

NC = 2
NS = 16
NW = NC * NS
CHUNK = 128
N_CHUNKS = N_EDGES // CHUNK
NPT = 624
TAIL = N_NODES - NPT * NS
BCHUNK = 80
NB_CHUNKS = N_NODES // BCHUNK
FB = D_FEAT // 16


def _shuffle_xor(v, shift):
    idx = jnp.reshape(
        jax.lax.iota(jnp.int32, 16) ^ jnp.int32(shift), (16, 1))
    return lax.gather(
        v, idx,
        dimension_numbers=lax.GatherDimensionNumbers(
            offset_dims=(), collapsed_slice_dims=(0,), start_index_map=(0,)),
        slice_sizes=(1,), mode=lax.GatherScatterMode.PROMISE_IN_BOUNDS)


def _mesh():
    return plsc.VectorSubcoreMesh(
        core_axis_name="c", subcore_axis_name="s", num_cores=NC,
        num_subcores=NS)


def _lane_bcast(wblk, l):
    idx = jnp.full((16, 1), l, jnp.int32)
    return lax.gather(
        wblk, idx,
        dimension_numbers=lax.GatherDimensionNumbers(
            offset_dims=(), collapsed_slice_dims=(0,), start_index_map=(0,)),
        slice_sizes=(1,), mode=lax.GatherScatterMode.PROMISE_IN_BOUNDS)


@functools.partial(
    pl.kernel,
    out_type=jax.ShapeDtypeStruct((NC, N_NODES, D_FEAT), jnp.float32),
    mesh=_mesh(),
    scratch_types=[
        pltpu.VMEM_SHARED((N_NODES, D_FEAT), jnp.float32),
        pltpu.VMEM((3, CHUNK, D_FEAT), jnp.float32),
        pltpu.VMEM((3, 2, CHUNK), jnp.int32),
        pltpu.VMEM((3, CHUNK), jnp.float32),
        pltpu.SemaphoreType.DMA,
        pltpu.SemaphoreType.DMA,
        pltpu.SemaphoreType.DMA,
    ],
)
def _phase_a(vals_hbm, edges_hbm, w_hbm, zeros_hbm, part_hbm,
             acc_sp, rows_v, e_v, w_v, semI, semG, semS):
    cid = lax.axis_index("c")
    sid = lax.axis_index("s")
    wid = sid * NC + cid

    pltpu.sync_copy(zeros_hbm.at[pl.ds(0, NPT)],
                    acc_sp.at[pl.ds(sid * NPT, NPT)])

    @pl.when(sid == NS - 1)
    def _zero_tail():
        pltpu.sync_copy(zeros_hbm.at[pl.ds(0, TAIL)],
                        acc_sp.at[pl.ds(NPT * NS, TAIL)])

    rem = N_CHUNKS - (N_CHUNKS // NW) * NW
    cnt = jnp.where(wid < rem, N_CHUNKS // NW + 1, N_CHUNKS // NW)

    def coff(k):
        return (wid + NW * k) * CHUNK

    def stage_idx(k, b):
        pltpu.async_copy(edges_hbm.at[:, pl.ds(coff(k), CHUNK)],
                         e_v.at[b], semI)
        pltpu.async_copy(w_hbm.at[pl.ds(coff(k), CHUNK)], w_v.at[b], semI)

    def wait_idx(k, b):
        pltpu.make_async_copy(edges_hbm.at[:, pl.ds(coff(k), CHUNK)],
                              e_v.at[b], semI).wait()
        pltpu.make_async_copy(w_hbm.at[pl.ds(coff(k), CHUNK)],
                              w_v.at[b], semI).wait()

    def start_gather(b):
        pltpu.async_copy(vals_hbm.at[e_v.at[b, 0]], rows_v.at[b], semG)

    def wait_gather(b):
        pltpu.make_async_copy(vals_hbm.at[e_v.at[b, 0]], rows_v.at[b],
                              semG).wait()

    def start_scatter(b):
        pltpu.async_copy(rows_v.at[b], acc_sp.at[e_v.at[b, 1]], semS,
                         add=True)

    def wait_scatter(b):
        pltpu.make_async_copy(rows_v.at[b], acc_sp.at[e_v.at[b, 1]],
                              semS).wait()

    stage_idx(0, 0)
    wait_idx(0, 0)
    start_gather(0)

    @pl.when(cnt > 1)
    def _prefetch1():
        stage_idx(1, 1)

    plsc.subcore_barrier()

    def chunk_body(k, carry):
        b3 = lax.rem(k, 3)

        def run(b, nb, pb):
            wait_gather(b)

            @pl.when(k + 1 < cnt)
            def _next_gather():
                wait_idx(k + 1, nb)
                start_gather(nb)

            def grp_body(g, c):
                base = g * 16
                wblk = w_v[b, pl.ds(base, 16)]
                for l in range(16):
                    e = base + l
                    wv = _lane_bcast(wblk, l)
                    for j in range(FB):
                        rows_v[b, e, pl.ds(j * 16, 16)] = (
                            rows_v[b, e, pl.ds(j * 16, 16)] * wv)
                return c

            lax.fori_loop(0, CHUNK // 16, grp_body, 0)

            @pl.when(k > 0)
            def _drain_prev_scatter():
                wait_scatter(pb)

            start_scatter(b)

            @pl.when(k + 2 < cnt)
            def _next_idx():
                stage_idx(k + 2, pb)

        @pl.when(b3 == 0)
        def _b0():
            run(0, 1, 2)

        @pl.when(b3 == 1)
        def _b1():
            run(1, 2, 0)

        @pl.when(b3 == 2)
        def _b2():
            run(2, 0, 1)

        return carry

    lax.fori_loop(0, cnt, chunk_body, 0)
    lb = lax.rem(cnt - 1, 3)
    for b in range(3):
        @pl.when(lb == b)
        def _drain_last(b=b):
            wait_scatter(b)

    plsc.subcore_barrier()
    pltpu.sync_copy(acc_sp.at[pl.ds(sid * NPT, NPT)],
                    part_hbm.at[cid, pl.ds(sid * NPT, NPT)])

    @pl.when(sid == NS - 1)
    def _dump_tail():
        pltpu.sync_copy(acc_sp.at[pl.ds(NPT * NS, TAIL)],
                        part_hbm.at[cid, pl.ds(NPT * NS, TAIL)])


@functools.partial(
    pl.kernel,
    out_type=jax.ShapeDtypeStruct((N_NODES, D_FEAT), jnp.float32),
    mesh=_mesh(),
    scratch_types=[
        pltpu.VMEM((2, BCHUNK, D_FEAT), jnp.float32),
        pltpu.VMEM((2, BCHUNK, D_FEAT), jnp.float32),
        pltpu.VMEM((2, BCHUNK, D_FEAT), jnp.float32),
        pltpu.SemaphoreType.DMA,
        pltpu.SemaphoreType.DMA,
    ],
)
def _phase_b(part_hbm, out_hbm, a_v, b_v, o_v, semL, semO):
    cid = lax.axis_index("c")
    sid = lax.axis_index("s")
    wid = sid * NC + cid

    rem = NB_CHUNKS - (NB_CHUNKS // NW) * NW
    cnt = jnp.where(wid < rem, NB_CHUNKS // NW + 1, NB_CHUNKS // NW)

    def boff(k):
        return (wid + NW * k) * BCHUNK

    def stage(k, b):
        pltpu.async_copy(part_hbm.at[0, pl.ds(boff(k), BCHUNK)], a_v.at[b],
                         semL)
        pltpu.async_copy(part_hbm.at[1, pl.ds(boff(k), BCHUNK)], b_v.at[b],
                         semL)

    def wait_stage(k, b):
        pltpu.make_async_copy(part_hbm.at[0, pl.ds(boff(k), BCHUNK)],
                              a_v.at[b], semL).wait()
        pltpu.make_async_copy(part_hbm.at[1, pl.ds(boff(k), BCHUNK)],
                              b_v.at[b], semL).wait()

    def start_store(k, b):
        pltpu.async_copy(o_v.at[b], out_hbm.at[pl.ds(boff(k), BCHUNK)],
                         semO)

    def wait_store(k, b):
        pltpu.make_async_copy(o_v.at[b], out_hbm.at[pl.ds(boff(k), BCHUNK)],
                              semO).wait()

    stage(0, 0)

    @pl.when(cnt > 1)
    def _prefetch1():
        stage(1, 1)

    def chunk_body(k, carry):
        b2 = lax.rem(k, 2)

        def run(b):
            @pl.when(k > 1)
            def _drain_store():
                wait_store(k - 2, b)

            wait_stage(k, b)

            def node_body(i, c):
                vs = [a_v[b, i, pl.ds(j * 16, 16)] +
                      b_v[b, i, pl.ds(j * 16, 16)] for j in range(FB)]
                m = vs[0]
                for j in range(1, FB):
                    m = jnp.maximum(m, vs[j])
                for sh in (8, 4, 2, 1):
                    m = jnp.maximum(m, _shuffle_xor(m, sh))
                es = [jnp.exp(v - m) for v in vs]
                s = es[0]
                for j in range(1, FB):
                    s = s + es[j]
                for sh in (8, 4, 2, 1):
                    s = s + _shuffle_xor(s, sh)
                r = 1.0 / s
                for j in range(FB):
                    o_v[b, i, pl.ds(j * 16, 16)] = es[j] * r
                return c

            lax.fori_loop(0, BCHUNK, node_body, 0)
            start_store(k, b)

            @pl.when(k + 2 < cnt)
            def _next_stage():
                stage(k + 2, b)

        @pl.when(b2 == 0)
        def _b0():
            run(0)

        @pl.when(b2 == 1)
        def _b1():
            run(1)

        return carry

    lax.fori_loop(0, cnt, chunk_body, 0)

    lb1 = lax.rem(cnt - 1, 2)
    for bb in range(2):
        @pl.when(lb1 == bb)
        def _d1(bb=bb):
            wait_store(cnt - 1, bb)

        @pl.when((cnt > 1) & (lax.rem(cnt - 2, 2) == bb))
        def _d2(bb=bb):
            wait_store(cnt - 2, bb)


def kernel(x, edge_index, edge_weight):
    src = edge_index[0].astype(jnp.int32)
    dst = edge_index[1].astype(jnp.int32)
    w = edge_weight.astype(jnp.float32)
    edges = jnp.stack([src, dst])
    zeros = jnp.zeros((NPT, D_FEAT), jnp.float32)
    vals = x
    for _ in range(NUM_LAYERS):
        part = _phase_a(vals, edges, w, zeros)
        vals = _phase_b(part)
    return vals[N_NODES - NUM_OUTPUTS:]
